# Initial kernel scaffold; baseline (speedup 1.0000x reference)
import functools

import jax
import jax.numpy as jnp
from jax import lax
from jax.experimental import pallas as pl
from jax.experimental.pallas import tpu as pltpu
from jax.experimental.pallas import tpu_sc as plsc

F32 = jnp.float32
NC = 2
NS = 16
NW = NC * NS
WIN = 128
LANES = 16


def _ceil_to(v, m):
    return (v + m - 1) // m * m


def _sc_mesh():
    return plsc.VectorSubcoreMesh(core_axis_name="c", subcore_axis_name="s")


def _deg_kernel(n1, nwin):
    stripe = n1 // NS

    @functools.partial(
        pl.kernel,
        out_type=jax.ShapeDtypeStruct((NC, n1), F32),
        mesh=_sc_mesh(),
        scratch_types=[
            pltpu.VMEM((nwin, WIN), jnp.int32),
            pltpu.VMEM((WIN,), F32),
            pltpu.VMEM((stripe,), F32),
            pltpu.VMEM_SHARED((n1,), F32),
        ],
    )
    def deg(dstp_hbm, out_hbm, idx_v, ones_v, zb_v, acc_sh):
        c = lax.axis_index("c")
        s = lax.axis_index("s")
        wid = s * NC + c

        @pl.loop(0, WIN, step=LANES)
        def _(i):
            ones_v[pl.ds(i, LANES)] = jnp.full((LANES,), 1.0, F32)

        @pl.loop(0, stripe, step=LANES)
        def _(i):
            zb_v[pl.ds(i, LANES)] = jnp.zeros((LANES,), F32)

        base = s * stripe
        pltpu.sync_copy(zb_v, acc_sh.at[pl.ds(base, stripe)])
        pltpu.sync_copy(dstp_hbm.at[wid], idx_v)
        plsc.subcore_barrier()

        @pl.loop(0, nwin)
        def _(j):
            pltpu.sync_copy(ones_v, acc_sh.at[idx_v.at[j]], add=True)

        plsc.subcore_barrier()
        pltpu.sync_copy(acc_sh.at[pl.ds(base, stripe)], zb_v)
        pltpu.sync_copy(zb_v, out_hbm.at[c].at[pl.ds(base, stripe)])

    return deg


def _hop_kernel(n2, cp, nwin):
    stripe = n2 // NS

    @functools.partial(
        pl.kernel,
        out_type=jax.ShapeDtypeStruct((NC, n2, cp), F32),
        mesh=_sc_mesh(),
        scratch_types=[
            pltpu.VMEM((nwin, WIN), jnp.int32),
            pltpu.VMEM((nwin, WIN), jnp.int32),
            pltpu.VMEM((WIN, cp), F32),
            pltpu.VMEM((stripe, cp), F32),
            pltpu.VMEM_SHARED((n2, cp), F32),
        ],
    )
    def hop(u_hbm, srcp_hbm, dstp_hbm, out_hbm, src_v, dst_v, rows_v, zb_v,
            acc_sh):
        c = lax.axis_index("c")
        s = lax.axis_index("s")
        wid = s * NC + c

        @pl.loop(0, stripe)
        def _(r):
            for c0 in range(0, cp, LANES):
                zb_v[r, pl.ds(c0, LANES)] = jnp.zeros((LANES,), F32)

        base = s * stripe
        pltpu.sync_copy(zb_v, acc_sh.at[pl.ds(base, stripe), :])
        pltpu.sync_copy(srcp_hbm.at[wid], src_v)
        pltpu.sync_copy(dstp_hbm.at[wid], dst_v)
        plsc.subcore_barrier()

        @pl.loop(0, nwin)
        def _(j):
            pltpu.sync_copy(u_hbm.at[src_v.at[j]], rows_v)
            pltpu.sync_copy(rows_v, acc_sh.at[dst_v.at[j]], add=True)

        plsc.subcore_barrier()
        pltpu.sync_copy(acc_sh.at[pl.ds(base, stripe), :], zb_v)
        pltpu.sync_copy(zb_v, out_hbm.at[c].at[pl.ds(base, stripe), :])

    return hop


def _matmul(x, w48, n, cp):
    def body(x_ref, w_ref, o_ref):
        o_ref[...] = jnp.dot(x_ref[...], w_ref[...],
                             preferred_element_type=F32)

    return pl.pallas_call(
        body, out_shape=jax.ShapeDtypeStruct((n, cp), F32))(x, w48)


def _scale_first(d0, d1, y, n, cp):
    def body(d0_ref, d1_ref, y_ref, u_ref, dinv_ref):
        deg = d0_ref[...] + d1_ref[...] + 1.0
        dinv = lax.rsqrt(deg)
        dinv_ref[...] = dinv
        u_ref[...] = y_ref[...] * dinv

    return pl.pallas_call(
        body,
        out_shape=(jax.ShapeDtypeStruct((n, cp), F32),
                   jax.ShapeDtypeStruct((n, 1), F32)))(d0, d1, y)


def _combine_mid(p0, p1, u1, dinv, n, cp):
    def body(p0_ref, p1_ref, u1_ref, dinv_ref, u2_ref):
        d = dinv_ref[...]
        u2_ref[...] = (p0_ref[...] + p1_ref[...] + u1_ref[...]) * (d * d)

    return pl.pallas_call(
        body, out_shape=jax.ShapeDtypeStruct((n, cp), F32))(p0, p1, u1, dinv)


def _finalize(q0, q1, u2, dinv, b2, n, c):
    def body(q0_ref, q1_ref, u2_ref, dinv_ref, b_ref, o_ref):
        h = (q0_ref[...] + q1_ref[...] + u2_ref[...]) * dinv_ref[...]
        logits = h[:, :c] + b_ref[...]
        m = jnp.max(logits, axis=1, keepdims=True)
        e = jnp.exp(logits - m)
        lse = jnp.log(jnp.sum(e, axis=1, keepdims=True)) + m
        o_ref[...] = logits - lse

    return pl.pallas_call(
        body, out_shape=jax.ShapeDtypeStruct((n, c), F32))(q0, q1, u2, dinv,
                                                           b2)


def kernel(x, edge_index, W, b):
    n, d = x.shape
    e = edge_index.shape[1]
    c = W.shape[1]
    cp = _ceil_to(c, LANES)

    ew = _ceil_to(e, NW * WIN) // NW
    nwin = ew // WIN
    ep = NW * ew
    n1 = _ceil_to(n + 16, NS * LANES)
    n2 = _ceil_to(n + 16, NS)

    src = edge_index[0]
    dst = edge_index[1]
    pad = ep - e
    srcp = jnp.concatenate(
        [src, jnp.zeros((pad,), jnp.int32)]).reshape(NW, nwin, WIN)
    trash = (n + (jnp.arange(pad, dtype=jnp.int32) % 16)).astype(jnp.int32)
    dstp = jnp.concatenate([dst, trash]).reshape(NW, nwin, WIN)

    w48 = jnp.pad(W, ((0, 0), (0, cp - c)))
    b2 = b.reshape(1, c)

    degp = _deg_kernel(n1, nwin)(dstp)
    y = _matmul(x, w48, n, cp)

    d0 = degp[0, :n].reshape(n, 1)
    d1 = degp[1, :n].reshape(n, 1)
    u1, dinv = _scale_first(d0, d1, y, n, cp)

    h1 = _hop_kernel(n2, cp, nwin)(u1, srcp, dstp)
    u2 = _combine_mid(h1[0, :n], h1[1, :n], u1, dinv, n, cp)

    h2 = _hop_kernel(n2, cp, nwin)(u2, srcp, dstp)
    return _finalize(h2[0, :n], h2[1, :n], u2, dinv, b2, n, c)


# speedup vs baseline: 22.0937x; 22.0937x over previous
import functools

import jax
import jax.numpy as jnp
from jax import lax
from jax.experimental import pallas as pl
from jax.experimental.pallas import tpu as pltpu
from jax.experimental.pallas import tpu_sc as plsc

F32 = jnp.float32
NC = 2
NS = 16
NW = NC * NS
WIN = 128
LANES = 16


def _ceil_to(v, m):
    return (v + m - 1) // m * m


def _sc_mesh():
    return plsc.VectorSubcoreMesh(core_axis_name="c", subcore_axis_name="s")


_SC_PARAMS = pltpu.CompilerParams(use_tc_tiling_on_sc=False)


def _deg_kernel(n1, nwin):
    stripe = n1 // NS

    @functools.partial(
        pl.kernel,
        out_type=jax.ShapeDtypeStruct((NC, n1), F32),
        mesh=_sc_mesh(),
        scratch_types=[
            pltpu.VMEM((nwin, WIN), jnp.int32),
            pltpu.VMEM((WIN,), F32),
            pltpu.VMEM((stripe,), F32),
            pltpu.VMEM_SHARED((n1,), F32),
        ],
        compiler_params=_SC_PARAMS,
    )
    def deg(dstp_hbm, out_hbm, idx_v, ones_v, zb_v, acc_sh):
        c = lax.axis_index("c")
        s = lax.axis_index("s")
        wid = s * NC + c

        @pl.loop(0, WIN, step=LANES)
        def _(i):
            ones_v[pl.ds(i, LANES)] = jnp.full((LANES,), 1.0, F32)

        @pl.loop(0, stripe, step=LANES)
        def _(i):
            zb_v[pl.ds(i, LANES)] = jnp.zeros((LANES,), F32)

        base = s * stripe
        pltpu.sync_copy(zb_v, acc_sh.at[pl.ds(base, stripe)])
        pltpu.sync_copy(dstp_hbm.at[wid], idx_v)
        plsc.subcore_barrier()

        @pl.loop(0, nwin)
        def _(j):
            pltpu.sync_copy(ones_v, acc_sh.at[idx_v.at[j]], add=True)

        plsc.subcore_barrier()
        pltpu.sync_copy(acc_sh.at[pl.ds(base, stripe)], zb_v)
        pltpu.sync_copy(zb_v, out_hbm.at[c].at[pl.ds(base, stripe)])

    return deg


def _hop_kernel(n2, cp, nwin):
    stripe = n2 // NS

    @functools.partial(
        pl.kernel,
        out_type=jax.ShapeDtypeStruct((NC, n2, cp), F32),
        mesh=_sc_mesh(),
        scratch_types=[
            pltpu.VMEM((nwin, WIN), jnp.int32),
            pltpu.VMEM((nwin, WIN), jnp.int32),
            pltpu.VMEM((WIN, cp), F32),
            pltpu.VMEM((stripe, cp), F32),
            pltpu.VMEM_SHARED((n2, cp), F32),
        ],
        compiler_params=_SC_PARAMS,
    )
    def hop(u_hbm, srcp_hbm, dstp_hbm, out_hbm, src_v, dst_v, rows_v, zb_v,
            acc_sh):
        c = lax.axis_index("c")
        s = lax.axis_index("s")
        wid = s * NC + c

        @pl.loop(0, stripe)
        def _(r):
            for c0 in range(0, cp, LANES):
                zb_v[r, pl.ds(c0, LANES)] = jnp.zeros((LANES,), F32)

        base = s * stripe
        pltpu.sync_copy(zb_v, acc_sh.at[pl.ds(base, stripe), :])
        pltpu.sync_copy(srcp_hbm.at[wid], src_v)
        pltpu.sync_copy(dstp_hbm.at[wid], dst_v)
        plsc.subcore_barrier()

        @pl.loop(0, nwin)
        def _(j):
            pltpu.sync_copy(u_hbm.at[src_v.at[j]], rows_v)
            pltpu.sync_copy(rows_v, acc_sh.at[dst_v.at[j]], add=True)

        plsc.subcore_barrier()
        pltpu.sync_copy(acc_sh.at[pl.ds(base, stripe), :], zb_v)
        pltpu.sync_copy(zb_v, out_hbm.at[c].at[pl.ds(base, stripe), :])

    return hop


def _matmul(x, w48, n, cp):
    def body(x_ref, w_ref, o_ref):
        o_ref[...] = jnp.dot(x_ref[...], w_ref[...],
                             preferred_element_type=F32)

    return pl.pallas_call(
        body, out_shape=jax.ShapeDtypeStruct((n, cp), F32))(x, w48)


def _scale_first(d0, d1, y, n, cp):
    def body(d0_ref, d1_ref, y_ref, u_ref, dinv_ref):
        deg = d0_ref[...] + d1_ref[...] + 1.0
        dinv = lax.rsqrt(deg)
        dinv_ref[...] = dinv
        u_ref[...] = y_ref[...] * dinv

    return pl.pallas_call(
        body,
        out_shape=(jax.ShapeDtypeStruct((n, cp), F32),
                   jax.ShapeDtypeStruct((n, 1), F32)))(d0, d1, y)


def _combine_mid(p0, p1, u1, dinv, n, cp):
    def body(p0_ref, p1_ref, u1_ref, dinv_ref, u2_ref):
        d = dinv_ref[...]
        u2_ref[...] = (p0_ref[...] + p1_ref[...] + u1_ref[...]) * (d * d)

    return pl.pallas_call(
        body, out_shape=jax.ShapeDtypeStruct((n, cp), F32))(p0, p1, u1, dinv)


def _finalize(q0, q1, u2, dinv, b2, n, c):
    def body(q0_ref, q1_ref, u2_ref, dinv_ref, b_ref, o_ref):
        h = (q0_ref[...] + q1_ref[...] + u2_ref[...]) * dinv_ref[...]
        logits = h[:, :c] + b_ref[...]
        m = jnp.max(logits, axis=1, keepdims=True)
        e = jnp.exp(logits - m)
        lse = jnp.log(jnp.sum(e, axis=1, keepdims=True)) + m
        o_ref[...] = logits - lse

    return pl.pallas_call(
        body, out_shape=jax.ShapeDtypeStruct((n, c), F32))(q0, q1, u2, dinv,
                                                           b2)


def kernel(x, edge_index, W, b):
    n, d = x.shape
    e = edge_index.shape[1]
    c = W.shape[1]
    cp = _ceil_to(c, LANES)

    ew = _ceil_to(e, NW * WIN) // NW
    nwin = ew // WIN
    ep = NW * ew
    n1 = _ceil_to(n + 16, NS * LANES)
    n2 = _ceil_to(n + 16, NS * 8)

    src = edge_index[0]
    dst = edge_index[1]
    pad = ep - e
    srcp = jnp.concatenate(
        [src, jnp.zeros((pad,), jnp.int32)]).reshape(NW, nwin, WIN)
    trash = (n + (jnp.arange(pad, dtype=jnp.int32) % 16)).astype(jnp.int32)
    dstp = jnp.concatenate([dst, trash]).reshape(NW, nwin, WIN)

    w48 = jnp.pad(W, ((0, 0), (0, cp - c)))
    b2 = b.reshape(1, c)

    degp = _deg_kernel(n1, nwin)(dstp)
    y = _matmul(x, w48, n, cp)

    d0 = degp[0, :n].reshape(n, 1)
    d1 = degp[1, :n].reshape(n, 1)
    u1, dinv = _scale_first(d0, d1, y, n, cp)

    h1 = _hop_kernel(n2, cp, nwin)(u1, srcp, dstp)
    u2 = _combine_mid(h1[0, :n], h1[1, :n], u1, dinv, n, cp)

    h2 = _hop_kernel(n2, cp, nwin)(u2, srcp, dstp)
    return _finalize(h2[0, :n], h2[1, :n], u2, dinv, b2, n, c)
